# baseline (speedup 1.0000x reference)
import functools

import jax
import jax.numpy as jnp
from jax.experimental import pallas as pl
from jax.experimental.pallas import tpu as pltpu

N = 4096
F_IN = 256
HID = 64
NHEAD = 4
NCLASS = 16
ALPHA = 0.2
TINY = 1e-30

BM = 256
BP = 512
F32 = jnp.float32
BF16 = jnp.bfloat16


def _leaky(x):
    return jnp.maximum(x, ALPHA * x)


def _elu(x):
    return jnp.where(x > 0, x, jnp.exp(x) - 1.0)


def _proj1_body(x_ref, w_ref, a1_ref, a2_ref, wh_ref, s1_ref, s2_ref,
                csum_ref):
    i = pl.program_id(0)
    xb = x_ref[...].astype(BF16)
    ones = jnp.ones((BP, 1), dtype=BF16)
    zeros = jnp.zeros((BP, HID - 1), dtype=BF16)
    for h in range(NHEAD):
        wh = jnp.dot(xb, w_ref[h].astype(BF16), preferred_element_type=F32)
        whaug = jnp.concatenate([wh.astype(BF16), ones, zeros], axis=1)
        wh_ref[h] = whaug
        s1_ref[h] = jnp.dot(wh, a1_ref[h], preferred_element_type=F32)
        s2_ref[h] = jnp.dot(wh, a2_ref[h], preferred_element_type=F32)
        cs = jnp.sum(whaug.astype(F32), axis=0, keepdims=True)

        @pl.when(i == 0)
        def _init():
            csum_ref[h] = cs

        @pl.when(i > 0)
        def _acc():
            csum_ref[h] = csum_ref[h] + cs


def _fused_attn_body(mask_ref, s1_ref, s2_ref, wh_ref, wo_ref, b1_ref, b2_ref,
                     csum_ref, out_ref, mask8_scr, gaug_scr, t1_scr,
                     t2row_scr, gsum_scr):
    phase = pl.program_id(0)
    i = pl.program_id(1)
    rows = pl.ds(i * BM, BM)

    @pl.when(phase == 0)
    def _layer1():
        mask8_scr[rows, :] = mask_ref[...].astype(jnp.int8)
        maskb = mask_ref[...].astype(BF16)
        g = jnp.zeros((BM, NCLASS), dtype=F32)
        for h in range(NHEAD):
            s2h = s2_ref[h]
            s2m = jnp.max(s2h)
            bb1 = jnp.exp(s2h - s2m).astype(BF16)
            bb2 = jnp.exp(ALPHA * (s2h - s2m)).astype(BF16)
            u = s1_ref[h] + s2m
            m = _leaky(u)
            aa1 = jnp.exp(u - m).astype(BF16)
            aa2 = jnp.exp(ALPHA * u - m).astype(BF16)
            p = jnp.maximum(aa1 * bb1, aa2 * bb2) * maskb
            acc = (jnp.dot(p, wh_ref[h], preferred_element_type=F32)
                   + TINY * csum_ref[h])
            hp = acc[:, :HID] / acc[:, HID:HID + 1]
            g = g + jnp.dot(_elu(hp), wo_ref[h], preferred_element_type=F32)
        ones = jnp.ones((BM, 1), dtype=BF16)
        zeros = jnp.zeros((BM, NCLASS - 1), dtype=BF16)
        gaug = jnp.concatenate([g.astype(BF16), ones, zeros], axis=1)
        gaug_scr[rows, :] = gaug
        gs = jnp.sum(gaug.astype(F32), axis=0, keepdims=True)

        @pl.when(i == 0)
        def _init():
            gsum_scr[...] = gs

        @pl.when(i > 0)
        def _acc():
            gsum_scr[...] = gsum_scr[...] + gs

        t1_scr[rows, :] = jnp.dot(g, b1_ref[...], preferred_element_type=F32)
        t2 = jnp.dot(g, b2_ref[...], preferred_element_type=F32)
        t2row_scr[:, rows] = t2.reshape(1, BM)

    @pl.when(phase == 1)
    def _layer2():
        maskb = mask8_scr[rows, :].astype(BF16)
        t2h = t2row_scr[...]
        t2m = jnp.max(t2h)
        dd1 = jnp.exp(t2h - t2m).astype(BF16)
        dd2 = jnp.exp(ALPHA * (t2h - t2m)).astype(BF16)
        u = t1_scr[rows, :] + t2m
        m = _leaky(u)
        cc1 = jnp.exp(u - m).astype(BF16)
        cc2 = jnp.exp(ALPHA * u - m).astype(BF16)
        p = jnp.maximum(cc1 * dd1, cc2 * dd2) * maskb
        acc = (jnp.dot(p, gaug_scr[...], preferred_element_type=F32)
               + TINY * gsum_scr[...])
        hp = acc[:, :NCLASS] / acc[:, NCLASS:NCLASS + 1]
        o = _elu(hp)
        mm = jnp.max(o, axis=1, keepdims=True)
        z = o - mm
        lse = jnp.log(jnp.sum(jnp.exp(z), axis=1, keepdims=True))
        out_ref[...] = z - lse


@functools.partial(jax.jit, static_argnames=())
def kernel(x, attn_mask, W0, a0, W1, a1, W2, a2, W3, a3, W_out, a_out):
    wstack = jnp.stack([W0, W1, W2, W3])
    a1stack = jnp.stack([a0[:HID], a1[:HID], a2[:HID], a3[:HID]])
    a2stack = jnp.stack([a0[HID:], a1[HID:], a2[HID:], a3[HID:]])
    wostack = W_out.reshape(NHEAD, HID, NCLASS)
    b1 = a_out[:NCLASS]
    b2 = a_out[NCLASS:]

    wh, s1, s2, csum = pl.pallas_call(
        _proj1_body,
        grid=(N // BP,),
        in_specs=[
            pl.BlockSpec((BP, F_IN), lambda i: (i, 0)),
            pl.BlockSpec((NHEAD, F_IN, HID), lambda i: (0, 0, 0)),
            pl.BlockSpec((NHEAD, HID, 1), lambda i: (0, 0, 0)),
            pl.BlockSpec((NHEAD, HID, 1), lambda i: (0, 0, 0)),
        ],
        out_specs=[
            pl.BlockSpec((NHEAD, BP, 2 * HID), lambda i: (0, i, 0)),
            pl.BlockSpec((NHEAD, BP, 1), lambda i: (0, i, 0)),
            pl.BlockSpec((NHEAD, BP, 1), lambda i: (0, i, 0)),
            pl.BlockSpec((NHEAD, 1, 2 * HID), lambda i: (0, 0, 0)),
        ],
        out_shape=[
            jax.ShapeDtypeStruct((NHEAD, N, 2 * HID), BF16),
            jax.ShapeDtypeStruct((NHEAD, N, 1), F32),
            jax.ShapeDtypeStruct((NHEAD, N, 1), F32),
            jax.ShapeDtypeStruct((NHEAD, 1, 2 * HID), F32),
        ],
    )(x, wstack, a1stack, a2stack)

    s2r = s2.reshape(NHEAD, 1, N)

    out = pl.pallas_call(
        _fused_attn_body,
        grid=(2, N // BM),
        in_specs=[
            pl.BlockSpec((BM, N), lambda p, i: (jnp.where(p == 0, i, 0), 0)),
            pl.BlockSpec((NHEAD, BM, 1),
                         lambda p, i: (0, jnp.where(p == 0, i, 0), 0)),
            pl.BlockSpec((NHEAD, 1, N), lambda p, i: (0, 0, 0)),
            pl.BlockSpec((NHEAD, N, 2 * HID), lambda p, i: (0, 0, 0)),
            pl.BlockSpec((NHEAD, HID, NCLASS), lambda p, i: (0, 0, 0)),
            pl.BlockSpec((NCLASS, 1), lambda p, i: (0, 0)),
            pl.BlockSpec((NCLASS, 1), lambda p, i: (0, 0)),
            pl.BlockSpec((NHEAD, 1, 2 * HID), lambda p, i: (0, 0, 0)),
        ],
        out_specs=pl.BlockSpec((BM, NCLASS), lambda p, i: (i, 0)),
        out_shape=jax.ShapeDtypeStruct((N, NCLASS), F32),
        scratch_shapes=[
            pltpu.VMEM((N, N), jnp.int8),
            pltpu.VMEM((N, 2 * NCLASS), BF16),
            pltpu.VMEM((N, 1), F32),
            pltpu.VMEM((1, N), F32),
            pltpu.VMEM((1, 2 * NCLASS), F32),
        ],
    )(attn_mask, s1, s2r, wh, wostack, b1, b2, csum)

    return out


# speedup vs baseline: 1.0504x; 1.0504x over previous
import functools

import jax
import jax.numpy as jnp
from jax.experimental import pallas as pl
from jax.experimental.pallas import tpu as pltpu

N = 4096
F_IN = 256
HID = 64
NHEAD = 4
NCLASS = 16
ALPHA = 0.2
TINY = 1e-30

BM = 256
BP = 512
F32 = jnp.float32
BF16 = jnp.bfloat16


def _leaky(x):
    return jnp.maximum(x, ALPHA * x)


def _elu(x):
    return jnp.where(x > 0, x, jnp.exp(x) - 1.0)


def _proj1_body(x_ref, w_ref, a1_ref, a2_ref, wh_ref, s1_ref, s2_ref):
    x = x_ref[...]
    ones = jnp.ones((BP, 1), dtype=BF16)
    zeros = jnp.zeros((BP, HID - 1), dtype=BF16)
    for h in range(NHEAD):
        wh = jnp.dot(x, w_ref[h], preferred_element_type=F32)
        wh_ref[h] = jnp.concatenate([wh.astype(BF16), ones, zeros], axis=1)
        s1_ref[h] = jnp.dot(wh, a1_ref[h], preferred_element_type=F32)
        s2_ref[h] = jnp.dot(wh, a2_ref[h], preferred_element_type=F32)


def _fused_attn_body(mask_ref, s1_ref, s2_ref, wh_ref, wo_ref, b1_ref, b2_ref,
                     out_ref, mask8_scr, gaug_scr, t1_scr, t2row_scr):
    phase = pl.program_id(0)
    i = pl.program_id(1)
    rows = pl.ds(i * BM, BM)

    @pl.when(phase == 0)
    def _layer1():
        valid = mask_ref[...] > 0
        mask8_scr[rows, :] = valid.astype(jnp.int8)
        g = jnp.zeros((BM, NCLASS), dtype=F32)
        for h in range(NHEAD):
            s2h = s2_ref[h]
            s2m = jnp.max(s2h)
            bb1 = jnp.exp(s2h - s2m).astype(BF16)
            bb2 = jnp.exp(ALPHA * (s2h - s2m)).astype(BF16)
            u = s1_ref[h] + s2m
            m = _leaky(u)
            aa1 = jnp.exp(u - m).astype(BF16)
            aa2 = jnp.exp(ALPHA * u - m).astype(BF16)
            p = jnp.where(valid, jnp.maximum(aa1 * bb1, aa2 * bb2), 0) + TINY
            acc = jnp.dot(p, wh_ref[h], preferred_element_type=F32)
            hp = acc[:, :HID] / acc[:, HID:HID + 1]
            g = g + jnp.dot(_elu(hp), wo_ref[h], preferred_element_type=F32)
        ones = jnp.ones((BM, 1), dtype=BF16)
        zeros = jnp.zeros((BM, NCLASS - 1), dtype=BF16)
        gaug_scr[rows, :] = jnp.concatenate([g.astype(BF16), ones, zeros],
                                            axis=1)
        t1_scr[rows, :] = jnp.dot(g, b1_ref[...], preferred_element_type=F32)
        t2 = jnp.dot(g, b2_ref[...], preferred_element_type=F32)
        t2row_scr[:, rows] = t2.reshape(1, BM)

    @pl.when(phase == 1)
    def _layer2():
        maskb = mask8_scr[rows, :].astype(BF16)
        t2h = t2row_scr[...]
        t2m = jnp.max(t2h)
        dd1 = jnp.exp(t2h - t2m).astype(BF16)
        dd2 = jnp.exp(ALPHA * (t2h - t2m)).astype(BF16)
        u = t1_scr[rows, :] + t2m
        m = _leaky(u)
        cc1 = jnp.exp(u - m).astype(BF16)
        cc2 = jnp.exp(ALPHA * u - m).astype(BF16)
        p = jnp.maximum(cc1 * dd1, cc2 * dd2) * maskb + TINY
        acc = jnp.dot(p, gaug_scr[...], preferred_element_type=F32)
        hp = acc[:, :NCLASS] / acc[:, NCLASS:NCLASS + 1]
        o = _elu(hp)
        mm = jnp.max(o, axis=1, keepdims=True)
        z = o - mm
        lse = jnp.log(jnp.sum(jnp.exp(z), axis=1, keepdims=True))
        out_ref[...] = z - lse


@functools.partial(jax.jit, static_argnames=())
def kernel(x, attn_mask, W0, a0, W1, a1, W2, a2, W3, a3, W_out, a_out):
    wstack = jnp.stack([W0, W1, W2, W3])
    a1stack = jnp.stack([a0[:HID], a1[:HID], a2[:HID], a3[:HID]])
    a2stack = jnp.stack([a0[HID:], a1[HID:], a2[HID:], a3[HID:]])
    wostack = W_out.reshape(NHEAD, HID, NCLASS)
    b1 = a_out[:NCLASS]
    b2 = a_out[NCLASS:]

    wh, s1, s2 = pl.pallas_call(
        _proj1_body,
        grid=(N // BP,),
        in_specs=[
            pl.BlockSpec((BP, F_IN), lambda i: (i, 0)),
            pl.BlockSpec((NHEAD, F_IN, HID), lambda i: (0, 0, 0)),
            pl.BlockSpec((NHEAD, HID, 1), lambda i: (0, 0, 0)),
            pl.BlockSpec((NHEAD, HID, 1), lambda i: (0, 0, 0)),
        ],
        out_specs=[
            pl.BlockSpec((NHEAD, BP, 2 * HID), lambda i: (0, i, 0)),
            pl.BlockSpec((NHEAD, BP, 1), lambda i: (0, i, 0)),
            pl.BlockSpec((NHEAD, BP, 1), lambda i: (0, i, 0)),
        ],
        out_shape=[
            jax.ShapeDtypeStruct((NHEAD, N, 2 * HID), BF16),
            jax.ShapeDtypeStruct((NHEAD, N, 1), F32),
            jax.ShapeDtypeStruct((NHEAD, N, 1), F32),
        ],
    )(x, wstack, a1stack, a2stack)

    s2r = s2.reshape(NHEAD, 1, N)

    out = pl.pallas_call(
        _fused_attn_body,
        grid=(2, N // BM),
        in_specs=[
            pl.BlockSpec((BM, N), lambda p, i: (jnp.where(p == 0, i, 0), 0)),
            pl.BlockSpec((NHEAD, BM, 1),
                         lambda p, i: (0, jnp.where(p == 0, i, 0), 0)),
            pl.BlockSpec((NHEAD, 1, N), lambda p, i: (0, 0, 0)),
            pl.BlockSpec((NHEAD, N, 2 * HID), lambda p, i: (0, 0, 0)),
            pl.BlockSpec((NHEAD, HID, NCLASS), lambda p, i: (0, 0, 0)),
            pl.BlockSpec((NCLASS, 1), lambda p, i: (0, 0)),
            pl.BlockSpec((NCLASS, 1), lambda p, i: (0, 0)),
        ],
        out_specs=pl.BlockSpec((BM, NCLASS), lambda p, i: (i, 0)),
        out_shape=jax.ShapeDtypeStruct((N, NCLASS), F32),
        scratch_shapes=[
            pltpu.VMEM((N, N), jnp.int8),
            pltpu.VMEM((N, 2 * NCLASS), BF16),
            pltpu.VMEM((N, 1), F32),
            pltpu.VMEM((1, N), F32),
        ],
    )(attn_mask, s1, s2r, wh, wostack, b1, b2)

    return out
